# baseline (speedup 1.0000x reference)
import functools

import jax
import jax.numpy as jnp
from jax import lax
from jax.experimental import pallas as pl
from jax.experimental.pallas import tpu as pltpu
from jax.experimental.pallas import tpu_sc as plsc

N = 10000
E = 320000
G = 500
ECG = 8000
DIN = 140
H = 128
FC = 396

NC = 2
NS = 16
NT = NC * NS
CH = 256
NP = 10240
CPT = 40
EPT = CPT * CH
EP = NT * EPT
RPT = NP // NS
GP = 512
BR = 512
GRID_N = NP // BR


def _deg_body(dst_hbm, zeros_hbm, ones_hbm, out_hbm, acc_sh, ones_v, idx_v):
    cid = lax.axis_index("c")
    sid = lax.axis_index("s")
    wid = cid * NS + sid
    r0 = sid * RPT
    pltpu.sync_copy(zeros_hbm.at[pl.ds(r0, RPT)], acc_sh.at[pl.ds(r0, RPT)])
    pltpu.sync_copy(ones_hbm, ones_v)
    plsc.subcore_barrier()
    base = wid * EPT

    def chunk(c, carry):
        pltpu.sync_copy(dst_hbm.at[pl.ds(base + c * CH, CH)], idx_v.at[0])
        pltpu.sync_copy(ones_v, acc_sh.at[idx_v.at[0]], add=True)
        return carry

    lax.fori_loop(0, CPT, chunk, 0)
    plsc.subcore_barrier()
    pltpu.sync_copy(acc_sh.at[pl.ds(r0, RPT)], out_hbm.at[cid, pl.ds(r0, RPT)])


def _mp_body(hp_hbm, src_hbm, dst_hbm, zeros_hbm, out_hbm,
             acc_sh, sidx_v, didx_v, rows_v, gsem):
    cid = lax.axis_index("c")
    sid = lax.axis_index("s")
    wid = cid * NS + sid
    r0 = sid * RPT
    pltpu.sync_copy(zeros_hbm.at[pl.ds(r0, RPT)], acc_sh.at[pl.ds(r0, RPT)])
    plsc.subcore_barrier()
    base = wid * EPT

    def chunk(c, carry):
        off = base + c * CH
        pltpu.sync_copy(src_hbm.at[pl.ds(off, CH)], sidx_v)
        pltpu.sync_copy(dst_hbm.at[pl.ds(off, CH)], didx_v.at[0])
        pltpu.async_copy(hp_hbm.at[sidx_v], rows_v, gsem).wait()
        pltpu.sync_copy(rows_v, acc_sh.at[didx_v.at[0]], add=True)
        return carry

    lax.fori_loop(0, CPT, chunk, 0)
    plsc.subcore_barrier()
    pltpu.sync_copy(acc_sh.at[pl.ds(r0, RPT)], out_hbm.at[cid, pl.ds(r0, RPT)])


def _make_sc_kernels():
    mesh = plsc.VectorSubcoreMesh(core_axis_name="c", subcore_axis_name="s")
    deg_k = pl.kernel(
        _deg_body,
        out_type=jax.ShapeDtypeStruct((NC, NP), jnp.float32),
        mesh=mesh,
        scratch_types=[
            pltpu.VMEM_SHARED((NP,), jnp.float32),
            pltpu.VMEM((CH,), jnp.float32),
            pltpu.VMEM((1, CH), jnp.int32),
        ],
    )
    mp_k = pl.kernel(
        _mp_body,
        out_type=jax.ShapeDtypeStruct((NC, NP, H), jnp.float32),
        mesh=mesh,
        scratch_types=[
            pltpu.VMEM_SHARED((NP, H), jnp.float32),
            pltpu.VMEM((CH,), jnp.int32),
            pltpu.VMEM((1, CH), jnp.int32),
            pltpu.VMEM((CH, H), jnp.float32),
            pltpu.SemaphoreType.DMA,
        ],
    )
    return deg_k, mp_k


def _dinv_block(degp_ref):
    deg = degp_ref[0] + degp_ref[1] + 1.0
    return lax.rsqrt(deg)


def _stage2_body(x_ref, w0_ref, degp_ref, h0p_ref):
    dinv = _dinv_block(degp_ref)
    h0 = jnp.dot(x_ref[...], w0_ref[...], preferred_element_type=jnp.float32)
    h0p_ref[...] = h0 * dinv


def _stage4_body(acc_ref, h0p_ref, degp_ref, b0_ref, w1_ref, h1_ref, h1p_ref):
    dinv = _dinv_block(degp_ref)
    pre = dinv * (acc_ref[0] + acc_ref[1] + h0p_ref[...]) + b0_ref[...]
    h1 = jnp.tanh(pre)
    h1_ref[...] = h1
    h1p_ref[...] = jnp.dot(h1, w1_ref[...],
                           preferred_element_type=jnp.float32) * dinv


def _stage6_body(x_ref, h1_ref, h1p_ref, acc_ref, degp_ref, b1_ref, batch_ref,
                 pooled_ref):
    i = pl.program_id(0)
    dinv = _dinv_block(degp_ref)
    pre = dinv * (acc_ref[0] + acc_ref[1] + h1p_ref[...]) + b1_ref[...]
    h2 = jnp.tanh(pre)
    xcat = jnp.concatenate([x_ref[...], h1_ref[...], h2], axis=1)
    cols = lax.broadcasted_iota(jnp.int32, (BR, GP), 1)
    oh = (jnp.broadcast_to(batch_ref[...], (BR, GP)) == cols).astype(jnp.float32)
    contrib = lax.dot_general(oh, xcat, (((0,), (0,)), ((), ())),
                              preferred_element_type=jnp.float32)

    @pl.when(i == 0)
    def _():
        pooled_ref[...] = jnp.zeros_like(pooled_ref)

    pooled_ref[...] += contrib


def _stage7_body(pooled_ref, wg_ref, atts_ref, attd_ref, bg_ref,
                 scg_ref, dcg_ref, wfa_ref, wfb_ref, bfc_ref, out_ref):
    pooled = pooled_ref[...]
    nblk = ECG // 500
    cols = lax.broadcasted_iota(jnp.int32, (500, GP), 1)
    c_mat = jnp.zeros((GP, GP), jnp.float32)
    for j in range(nblk):
        sb = scg_ref[pl.ds(j * 500, 500), :]
        db = dcg_ref[pl.ds(j * 500, 500), :]
        oh_s = (jnp.broadcast_to(sb, (500, GP)) == cols).astype(jnp.bfloat16)
        oh_d = (jnp.broadcast_to(db, (500, GP)) == cols).astype(jnp.bfloat16)
        c_mat += lax.dot_general(oh_d, oh_s, (((0,), (0,)), ((), ())),
                                 preferred_element_type=jnp.float32)
    rows_g = lax.broadcasted_iota(jnp.int32, (GP, GP), 0)
    cols_g = lax.broadcasted_iota(jnp.int32, (GP, GP), 1)
    c_mat += jnp.where((rows_g == cols_g) & (rows_g < G), 1.0, 0.0)

    hg = jnp.dot(pooled, wg_ref[...], preferred_element_type=jnp.float32)
    a_s = lax.dot_general(atts_ref[...], hg, (((1,), (1,)), ((), ())),
                          preferred_element_type=jnp.float32)
    a_d = lax.dot_general(hg, attd_ref[...], (((1,), (1,)), ((), ())),
                          preferred_element_type=jnp.float32)
    e_mat = a_d + a_s
    e_mat = jnp.where(e_mat >= 0.0, e_mat, 0.2 * e_mat)
    emax = jnp.max(jnp.where(c_mat > 0.0, e_mat, -1e30), axis=1, keepdims=True)
    emax = jnp.where(emax > -1e29, emax, 0.0)
    ex = c_mat * jnp.exp(e_mat - emax)
    den = jnp.sum(ex, axis=1, keepdims=True)
    alpha = ex / jnp.maximum(den, 1e-16)
    ctx = jnp.tanh(jnp.dot(alpha, hg, preferred_element_type=jnp.float32)
                   + bg_ref[...])
    out_ref[...] = (jnp.dot(pooled, wfa_ref[...],
                            preferred_element_type=jnp.float32)
                    + jnp.dot(ctx, wfb_ref[...],
                              preferred_element_type=jnp.float32)
                    + bfc_ref[...])


def _row_spec(cols):
    return pl.BlockSpec((BR, cols), lambda i: (i, 0))


def _fixed_spec(shape):
    nd = len(shape)
    return pl.BlockSpec(shape, lambda i, _nd=nd: (0,) * _nd)


_DEGP_SPEC = pl.BlockSpec((NC, BR, 1), lambda i: (0, i, 0))
_ACC_SPEC = pl.BlockSpec((NC, BR, H), lambda i: (0, i, 0))


def kernel(x, edge_index, batch, edge_index_cg, W0, b0, W1, b1, Wg,
           att_src, att_dst, bg, Wfc, bfc):
    src = edge_index[0].astype(jnp.int32)
    dst = edge_index[1].astype(jnp.int32)
    pad_e = EP - E
    srcp = jnp.concatenate([src, jnp.full((pad_e,), N, jnp.int32)])
    dstp = jnp.concatenate([dst, jnp.full((pad_e,), N, jnp.int32)])
    xp = jnp.pad(x, ((0, NP - N), (0, 0)))
    batchp = jnp.concatenate(
        [batch.astype(jnp.int32), jnp.full((NP - N,), GP - 1, jnp.int32)]
    ).reshape(NP, 1)
    zeros_mp = jnp.zeros((NP, H), jnp.float32)
    zeros_dg = jnp.zeros((NP,), jnp.float32)
    ones_dg = jnp.ones((CH,), jnp.float32)
    scg = edge_index_cg[0].astype(jnp.int32).reshape(ECG, 1)
    dcg = edge_index_cg[1].astype(jnp.int32).reshape(ECG, 1)
    b0r = b0.reshape(1, H)
    b1r = b1.reshape(1, H)
    bgr = bg.reshape(1, FC)
    bfcr = bfc.reshape(1, FC)
    attsr = att_src.reshape(1, FC)
    attdr = att_dst.reshape(1, FC)
    wfa = Wfc[:FC]
    wfb = Wfc[FC:]

    deg_k, mp_k = _make_sc_kernels()

    degp = deg_k(dstp, zeros_dg, ones_dg).reshape(NC, NP, 1)

    h0p = pl.pallas_call(
        _stage2_body,
        grid=(GRID_N,),
        in_specs=[_row_spec(DIN), _fixed_spec((DIN, H)), _DEGP_SPEC],
        out_specs=_row_spec(H),
        out_shape=jax.ShapeDtypeStruct((NP, H), jnp.float32),
    )(xp, W0, degp)

    acc1 = mp_k(h0p, srcp, dstp, zeros_mp)

    h1, h1p = pl.pallas_call(
        _stage4_body,
        grid=(GRID_N,),
        in_specs=[_ACC_SPEC, _row_spec(H), _DEGP_SPEC,
                  _fixed_spec((1, H)), _fixed_spec((H, H))],
        out_specs=[_row_spec(H), _row_spec(H)],
        out_shape=[jax.ShapeDtypeStruct((NP, H), jnp.float32),
                   jax.ShapeDtypeStruct((NP, H), jnp.float32)],
    )(acc1, h0p, degp, b0r, W1)

    acc2 = mp_k(h1p, srcp, dstp, zeros_mp)

    pooled = pl.pallas_call(
        _stage6_body,
        grid=(GRID_N,),
        in_specs=[_row_spec(DIN), _row_spec(H), _row_spec(H), _ACC_SPEC,
                  _DEGP_SPEC, _fixed_spec((1, H)), _row_spec(1)],
        out_specs=_fixed_spec((GP, FC)),
        out_shape=jax.ShapeDtypeStruct((GP, FC), jnp.float32),
    )(xp, h1, h1p, acc2, degp, b1r, batchp)

    outp = pl.pallas_call(
        _stage7_body,
        out_shape=jax.ShapeDtypeStruct((GP, FC), jnp.float32),
    )(pooled, Wg, attsr, attdr, bgr, scg, dcg, wfa, wfb, bfcr)

    return outp[:G]


# speedup vs baseline: 1.6157x; 1.6157x over previous
import functools

import jax
import jax.numpy as jnp
from jax import lax
from jax.experimental import pallas as pl
from jax.experimental.pallas import tpu as pltpu
from jax.experimental.pallas import tpu_sc as plsc

N = 10000
E = 320000
G = 500
ECG = 8000
DIN = 140
H = 128
FC = 396

NC = 2
NS = 16
NT = NC * NS
CH = 128
NP = 10240
CPT = 79
CPT0 = 90
CPT1 = 68
EPT = CPT * CH
EP = NT * EPT
RPT = NP // NS
GP = 512
BR = 512
GRID_N = NP // BR


def _deg_body(dst_hbm, zeros_hbm, ones_hbm, out_hbm, acc_sh, ones_v, idx_v):
    cid = lax.axis_index("c")
    sid = lax.axis_index("s")
    wid = cid * NS + sid
    r0 = sid * RPT
    pltpu.sync_copy(zeros_hbm.at[pl.ds(r0, RPT)], acc_sh.at[pl.ds(r0, RPT)])
    pltpu.sync_copy(ones_hbm, ones_v)
    plsc.subcore_barrier()
    base = wid * EPT

    def chunk(c, carry):
        pltpu.sync_copy(dst_hbm.at[pl.ds(base + c * CH, CH)], idx_v.at[0])
        pltpu.sync_copy(ones_v, acc_sh.at[idx_v.at[0]], add=True)
        return carry

    lax.fori_loop(0, CPT, chunk, 0)
    plsc.subcore_barrier()
    pltpu.sync_copy(acc_sh.at[pl.ds(r0, RPT)], out_hbm.at[cid, pl.ds(r0, RPT)])


def _mp_body(hp_hbm, src_hbm, dst_hbm, zeros_hbm, out_hbm,
             acc_sh, sidx_v, didx_v, rows_v, gsem):
    cid = lax.axis_index("c")
    sid = lax.axis_index("s")
    r0 = sid * RPT
    pltpu.sync_copy(zeros_hbm.at[pl.ds(r0, RPT)], acc_sh.at[pl.ds(r0, RPT)])
    plsc.subcore_barrier()
    cpt = jnp.where(cid == 0, CPT0, CPT1)
    base = jnp.where(cid == 0, sid * CPT0, NS * CPT0 + sid * CPT1) * CH

    def chunk(c, carry):
        off = base + c * CH
        pltpu.sync_copy(src_hbm.at[pl.ds(off, CH)], sidx_v)
        pltpu.sync_copy(dst_hbm.at[pl.ds(off, CH)], didx_v.at[0])
        pltpu.async_copy(hp_hbm.at[sidx_v], rows_v, gsem).wait()
        pltpu.sync_copy(rows_v, acc_sh.at[didx_v.at[0]], add=True)
        return carry

    lax.fori_loop(0, cpt, chunk, 0)
    plsc.subcore_barrier()
    pltpu.sync_copy(acc_sh.at[pl.ds(r0, RPT)], out_hbm.at[cid, pl.ds(r0, RPT)])


def _make_sc_kernels():
    mesh = plsc.VectorSubcoreMesh(core_axis_name="c", subcore_axis_name="s")
    deg_k = pl.kernel(
        _deg_body,
        out_type=jax.ShapeDtypeStruct((NC, NP), jnp.float32),
        mesh=mesh,
        scratch_types=[
            pltpu.VMEM_SHARED((NP,), jnp.float32),
            pltpu.VMEM((CH,), jnp.float32),
            pltpu.VMEM((1, CH), jnp.int32),
        ],
    )
    mp_k = pl.kernel(
        _mp_body,
        out_type=jax.ShapeDtypeStruct((NC, NP, H), jnp.float32),
        mesh=mesh,
        scratch_types=[
            pltpu.VMEM_SHARED((NP, H), jnp.float32),
            pltpu.VMEM((CH,), jnp.int32),
            pltpu.VMEM((1, CH), jnp.int32),
            pltpu.VMEM((CH, H), jnp.float32),
            pltpu.SemaphoreType.DMA,
        ],
    )
    return deg_k, mp_k


def _dinv_block(degp_ref):
    deg = degp_ref[0] + degp_ref[1] + 1.0
    return lax.rsqrt(deg)


def _stage2_body(x_ref, w0_ref, degp_ref, h0p_ref):
    dinv = _dinv_block(degp_ref)
    h0 = jnp.dot(x_ref[...], w0_ref[...], preferred_element_type=jnp.float32)
    h0p_ref[...] = h0 * dinv


def _stage4_body(acc_ref, h0p_ref, degp_ref, b0_ref, w1_ref, h1_ref, h1p_ref):
    dinv = _dinv_block(degp_ref)
    pre = dinv * (acc_ref[0] + acc_ref[1] + h0p_ref[...]) + b0_ref[...]
    h1 = jnp.tanh(pre)
    h1_ref[...] = h1
    h1p_ref[...] = jnp.dot(h1, w1_ref[...],
                           preferred_element_type=jnp.float32) * dinv


def _stage6_body(x_ref, h1_ref, h1p_ref, acc_ref, degp_ref, b1_ref, batch_ref,
                 pooled_ref):
    i = pl.program_id(0)
    dinv = _dinv_block(degp_ref)
    pre = dinv * (acc_ref[0] + acc_ref[1] + h1p_ref[...]) + b1_ref[...]
    h2 = jnp.tanh(pre)
    xcat = jnp.concatenate([x_ref[...], h1_ref[...], h2], axis=1)
    cols = lax.broadcasted_iota(jnp.int32, (BR, GP), 1)
    oh = (jnp.broadcast_to(batch_ref[...], (BR, GP)) == cols).astype(jnp.float32)
    contrib = lax.dot_general(oh, xcat, (((0,), (0,)), ((), ())),
                              preferred_element_type=jnp.float32)

    @pl.when(i == 0)
    def _():
        pooled_ref[...] = jnp.zeros_like(pooled_ref)

    pooled_ref[...] += contrib


def _stage7_body(pooled_ref, wg_ref, atts_ref, attd_ref, bg_ref,
                 scg_ref, dcg_ref, wfa_ref, wfb_ref, bfc_ref, out_ref):
    pooled = pooled_ref[...]
    nblk = ECG // 500
    cols = lax.broadcasted_iota(jnp.int32, (500, GP), 1)
    c_mat = jnp.zeros((GP, GP), jnp.float32)
    for j in range(nblk):
        sb = scg_ref[pl.ds(j * 500, 500), :]
        db = dcg_ref[pl.ds(j * 500, 500), :]
        oh_s = (jnp.broadcast_to(sb, (500, GP)) == cols).astype(jnp.bfloat16)
        oh_d = (jnp.broadcast_to(db, (500, GP)) == cols).astype(jnp.bfloat16)
        c_mat += lax.dot_general(oh_d, oh_s, (((0,), (0,)), ((), ())),
                                 preferred_element_type=jnp.float32)
    rows_g = lax.broadcasted_iota(jnp.int32, (GP, GP), 0)
    cols_g = lax.broadcasted_iota(jnp.int32, (GP, GP), 1)
    c_mat += jnp.where((rows_g == cols_g) & (rows_g < G), 1.0, 0.0)

    hg = jnp.dot(pooled, wg_ref[...], preferred_element_type=jnp.float32)
    a_s = lax.dot_general(atts_ref[...], hg, (((1,), (1,)), ((), ())),
                          preferred_element_type=jnp.float32)
    a_d = lax.dot_general(hg, attd_ref[...], (((1,), (1,)), ((), ())),
                          preferred_element_type=jnp.float32)
    e_mat = a_d + a_s
    e_mat = jnp.where(e_mat >= 0.0, e_mat, 0.2 * e_mat)
    emax = jnp.max(jnp.where(c_mat > 0.0, e_mat, -1e30), axis=1, keepdims=True)
    emax = jnp.where(emax > -1e29, emax, 0.0)
    ex = c_mat * jnp.exp(e_mat - emax)
    den = jnp.sum(ex, axis=1, keepdims=True)
    alpha = ex / jnp.maximum(den, 1e-16)
    ctx = jnp.tanh(jnp.dot(alpha, hg, preferred_element_type=jnp.float32)
                   + bg_ref[...])
    out_ref[...] = (jnp.dot(pooled, wfa_ref[...],
                            preferred_element_type=jnp.float32)
                    + jnp.dot(ctx, wfb_ref[...],
                              preferred_element_type=jnp.float32)
                    + bfc_ref[...])


def _row_spec(cols):
    return pl.BlockSpec((BR, cols), lambda i: (i, 0))


def _fixed_spec(shape):
    nd = len(shape)
    return pl.BlockSpec(shape, lambda i, _nd=nd: (0,) * _nd)


_DEGP_SPEC = pl.BlockSpec((NC, BR, 1), lambda i: (0, i, 0))
_ACC_SPEC = pl.BlockSpec((NC, BR, H), lambda i: (0, i, 0))


def kernel(x, edge_index, batch, edge_index_cg, W0, b0, W1, b1, Wg,
           att_src, att_dst, bg, Wfc, bfc):
    src = edge_index[0].astype(jnp.int32)
    dst = edge_index[1].astype(jnp.int32)
    pad_e = EP - E
    srcp = jnp.concatenate([src, jnp.full((pad_e,), N, jnp.int32)])
    dstp = jnp.concatenate([dst, jnp.full((pad_e,), N, jnp.int32)])
    xp = jnp.pad(x, ((0, NP - N), (0, 0)))
    batchp = jnp.concatenate(
        [batch.astype(jnp.int32), jnp.full((NP - N,), GP - 1, jnp.int32)]
    ).reshape(NP, 1)
    zeros_mp = jnp.zeros((NP, H), jnp.float32)
    zeros_dg = jnp.zeros((NP,), jnp.float32)
    ones_dg = jnp.ones((CH,), jnp.float32)
    scg = edge_index_cg[0].astype(jnp.int32).reshape(ECG, 1)
    dcg = edge_index_cg[1].astype(jnp.int32).reshape(ECG, 1)
    b0r = b0.reshape(1, H)
    b1r = b1.reshape(1, H)
    bgr = bg.reshape(1, FC)
    bfcr = bfc.reshape(1, FC)
    attsr = att_src.reshape(1, FC)
    attdr = att_dst.reshape(1, FC)
    wfa = Wfc[:FC]
    wfb = Wfc[FC:]

    deg_k, mp_k = _make_sc_kernels()

    degp = deg_k(dstp, zeros_dg, ones_dg).reshape(NC, NP, 1)

    h0p = pl.pallas_call(
        _stage2_body,
        grid=(GRID_N,),
        in_specs=[_row_spec(DIN), _fixed_spec((DIN, H)), _DEGP_SPEC],
        out_specs=_row_spec(H),
        out_shape=jax.ShapeDtypeStruct((NP, H), jnp.float32),
    )(xp, W0, degp)

    acc1 = mp_k(h0p, srcp, dstp, zeros_mp)

    h1, h1p = pl.pallas_call(
        _stage4_body,
        grid=(GRID_N,),
        in_specs=[_ACC_SPEC, _row_spec(H), _DEGP_SPEC,
                  _fixed_spec((1, H)), _fixed_spec((H, H))],
        out_specs=[_row_spec(H), _row_spec(H)],
        out_shape=[jax.ShapeDtypeStruct((NP, H), jnp.float32),
                   jax.ShapeDtypeStruct((NP, H), jnp.float32)],
    )(acc1, h0p, degp, b0r, W1)

    acc2 = mp_k(h1p, srcp, dstp, zeros_mp)

    pooled = pl.pallas_call(
        _stage6_body,
        grid=(GRID_N,),
        in_specs=[_row_spec(DIN), _row_spec(H), _row_spec(H), _ACC_SPEC,
                  _DEGP_SPEC, _fixed_spec((1, H)), _row_spec(1)],
        out_specs=_fixed_spec((GP, FC)),
        out_shape=jax.ShapeDtypeStruct((GP, FC), jnp.float32),
    )(xp, h1, h1p, acc2, degp, b1r, batchp)

    outp = pl.pallas_call(
        _stage7_body,
        out_shape=jax.ShapeDtypeStruct((GP, FC), jnp.float32),
    )(pooled, Wg, attsr, attdr, bgr, scg, dcg, wfa, wfb, bfcr)

    return outp[:G]


# speedup vs baseline: 1.6904x; 1.0463x over previous
import functools

import jax
import jax.numpy as jnp
from jax import lax
from jax.experimental import pallas as pl
from jax.experimental.pallas import tpu as pltpu
from jax.experimental.pallas import tpu_sc as plsc

N = 10000
E = 320000
G = 500
ECG = 8000
DIN = 140
H = 128
FC = 396

NC = 2
NS = 16
NT = NC * NS
CH = 128
NP = 10240
CPT = 79
CPT0 = 98
CPT1 = 60
EPT = CPT * CH
EP = NT * EPT
RPT = NP // NS
GP = 512
BR = 512
GRID_N = NP // BR


def _deg_body(dst_hbm, zeros_hbm, ones_hbm, out_hbm, acc_sh, ones_v, idx_v):
    cid = lax.axis_index("c")
    sid = lax.axis_index("s")
    wid = cid * NS + sid
    r0 = sid * RPT
    pltpu.sync_copy(zeros_hbm.at[pl.ds(r0, RPT)], acc_sh.at[pl.ds(r0, RPT)])
    pltpu.sync_copy(ones_hbm, ones_v)
    plsc.subcore_barrier()
    base = wid * EPT

    def chunk(c, carry):
        pltpu.sync_copy(dst_hbm.at[pl.ds(base + c * CH, CH)], idx_v.at[0])
        pltpu.sync_copy(ones_v, acc_sh.at[idx_v.at[0]], add=True)
        return carry

    lax.fori_loop(0, CPT, chunk, 0)
    plsc.subcore_barrier()
    pltpu.sync_copy(acc_sh.at[pl.ds(r0, RPT)], out_hbm.at[cid, pl.ds(r0, RPT)])


def _mp_body(hp_hbm, src_hbm, dst_hbm, zeros_hbm, out_hbm,
             acc_sh, sidx_v, didx_v, rows_v, gsem):
    cid = lax.axis_index("c")
    sid = lax.axis_index("s")
    r0 = sid * RPT
    pltpu.sync_copy(zeros_hbm.at[pl.ds(r0, RPT)], acc_sh.at[pl.ds(r0, RPT)])
    plsc.subcore_barrier()
    cpt = jnp.where(cid == 0, CPT0, CPT1)
    base = jnp.where(cid == 0, sid * CPT0, NS * CPT0 + sid * CPT1) * CH

    def chunk(c, carry):
        off = base + c * CH
        pltpu.sync_copy(src_hbm.at[pl.ds(off, CH)], sidx_v)
        pltpu.sync_copy(dst_hbm.at[pl.ds(off, CH)], didx_v.at[0])
        pltpu.async_copy(hp_hbm.at[sidx_v], rows_v, gsem).wait()
        pltpu.sync_copy(rows_v, acc_sh.at[didx_v.at[0]], add=True)
        return carry

    lax.fori_loop(0, cpt, chunk, 0)
    plsc.subcore_barrier()
    pltpu.sync_copy(acc_sh.at[pl.ds(r0, RPT)], out_hbm.at[cid, pl.ds(r0, RPT)])


def _make_sc_kernels():
    mesh = plsc.VectorSubcoreMesh(core_axis_name="c", subcore_axis_name="s")
    deg_k = pl.kernel(
        _deg_body,
        out_type=jax.ShapeDtypeStruct((NC, NP), jnp.float32),
        mesh=mesh,
        scratch_types=[
            pltpu.VMEM_SHARED((NP,), jnp.float32),
            pltpu.VMEM((CH,), jnp.float32),
            pltpu.VMEM((1, CH), jnp.int32),
        ],
    )
    mp_k = pl.kernel(
        _mp_body,
        out_type=jax.ShapeDtypeStruct((NC, NP, H), jnp.float32),
        mesh=mesh,
        scratch_types=[
            pltpu.VMEM_SHARED((NP, H), jnp.float32),
            pltpu.VMEM((CH,), jnp.int32),
            pltpu.VMEM((1, CH), jnp.int32),
            pltpu.VMEM((CH, H), jnp.float32),
            pltpu.SemaphoreType.DMA,
        ],
    )
    return deg_k, mp_k


def _dinv_block(degp_ref):
    deg = degp_ref[0] + degp_ref[1] + 1.0
    return lax.rsqrt(deg)


def _stage2_body(x_ref, w0_ref, degp_ref, h0p_ref):
    dinv = _dinv_block(degp_ref)
    h0 = jnp.dot(x_ref[...], w0_ref[...], preferred_element_type=jnp.float32)
    h0p_ref[...] = h0 * dinv


def _stage4_body(acc_ref, h0p_ref, degp_ref, b0_ref, w1_ref, h1_ref, h1p_ref):
    dinv = _dinv_block(degp_ref)
    pre = dinv * (acc_ref[0] + acc_ref[1] + h0p_ref[...]) + b0_ref[...]
    h1 = jnp.tanh(pre)
    h1_ref[...] = h1
    h1p_ref[...] = jnp.dot(h1, w1_ref[...],
                           preferred_element_type=jnp.float32) * dinv


def _stage6_body(x_ref, h1_ref, h1p_ref, acc_ref, degp_ref, b1_ref, batch_ref,
                 pooled_ref):
    i = pl.program_id(0)
    dinv = _dinv_block(degp_ref)
    pre = dinv * (acc_ref[0] + acc_ref[1] + h1p_ref[...]) + b1_ref[...]
    h2 = jnp.tanh(pre)
    xcat = jnp.concatenate([x_ref[...], h1_ref[...], h2], axis=1)
    cols = lax.broadcasted_iota(jnp.int32, (BR, GP), 1)
    oh = (jnp.broadcast_to(batch_ref[...], (BR, GP)) == cols).astype(jnp.float32)
    contrib = lax.dot_general(oh, xcat, (((0,), (0,)), ((), ())),
                              preferred_element_type=jnp.float32)

    @pl.when(i == 0)
    def _():
        pooled_ref[...] = jnp.zeros_like(pooled_ref)

    pooled_ref[...] += contrib


def _stage7_body(pooled_ref, wg_ref, atts_ref, attd_ref, bg_ref,
                 scg_ref, dcg_ref, wfa_ref, wfb_ref, bfc_ref, out_ref):
    pooled = pooled_ref[...]
    nblk = ECG // 500
    cols = lax.broadcasted_iota(jnp.int32, (500, GP), 1)
    c_mat = jnp.zeros((GP, GP), jnp.float32)
    for j in range(nblk):
        sb = scg_ref[pl.ds(j * 500, 500), :]
        db = dcg_ref[pl.ds(j * 500, 500), :]
        oh_s = (jnp.broadcast_to(sb, (500, GP)) == cols).astype(jnp.bfloat16)
        oh_d = (jnp.broadcast_to(db, (500, GP)) == cols).astype(jnp.bfloat16)
        c_mat += lax.dot_general(oh_d, oh_s, (((0,), (0,)), ((), ())),
                                 preferred_element_type=jnp.float32)
    rows_g = lax.broadcasted_iota(jnp.int32, (GP, GP), 0)
    cols_g = lax.broadcasted_iota(jnp.int32, (GP, GP), 1)
    c_mat += jnp.where((rows_g == cols_g) & (rows_g < G), 1.0, 0.0)

    hg = jnp.dot(pooled, wg_ref[...], preferred_element_type=jnp.float32)
    a_s = lax.dot_general(atts_ref[...], hg, (((1,), (1,)), ((), ())),
                          preferred_element_type=jnp.float32)
    a_d = lax.dot_general(hg, attd_ref[...], (((1,), (1,)), ((), ())),
                          preferred_element_type=jnp.float32)
    e_mat = a_d + a_s
    e_mat = jnp.where(e_mat >= 0.0, e_mat, 0.2 * e_mat)
    emax = jnp.max(jnp.where(c_mat > 0.0, e_mat, -1e30), axis=1, keepdims=True)
    emax = jnp.where(emax > -1e29, emax, 0.0)
    ex = c_mat * jnp.exp(e_mat - emax)
    den = jnp.sum(ex, axis=1, keepdims=True)
    alpha = ex / jnp.maximum(den, 1e-16)
    ctx = jnp.tanh(jnp.dot(alpha, hg, preferred_element_type=jnp.float32)
                   + bg_ref[...])
    out_ref[...] = (jnp.dot(pooled, wfa_ref[...],
                            preferred_element_type=jnp.float32)
                    + jnp.dot(ctx, wfb_ref[...],
                              preferred_element_type=jnp.float32)
                    + bfc_ref[...])


def _row_spec(cols):
    return pl.BlockSpec((BR, cols), lambda i: (i, 0))


def _fixed_spec(shape):
    nd = len(shape)
    return pl.BlockSpec(shape, lambda i, _nd=nd: (0,) * _nd)


_DEGP_SPEC = pl.BlockSpec((NC, BR, 1), lambda i: (0, i, 0))
_ACC_SPEC = pl.BlockSpec((NC, BR, H), lambda i: (0, i, 0))


def kernel(x, edge_index, batch, edge_index_cg, W0, b0, W1, b1, Wg,
           att_src, att_dst, bg, Wfc, bfc):
    src = edge_index[0].astype(jnp.int32)
    dst = edge_index[1].astype(jnp.int32)
    pad_e = EP - E
    srcp = jnp.concatenate([src, jnp.full((pad_e,), N, jnp.int32)])
    dstp = jnp.concatenate([dst, jnp.full((pad_e,), N, jnp.int32)])
    xp = jnp.pad(x, ((0, NP - N), (0, 0)))
    batchp = jnp.concatenate(
        [batch.astype(jnp.int32), jnp.full((NP - N,), GP - 1, jnp.int32)]
    ).reshape(NP, 1)
    zeros_mp = jnp.zeros((NP, H), jnp.float32)
    zeros_dg = jnp.zeros((NP,), jnp.float32)
    ones_dg = jnp.ones((CH,), jnp.float32)
    scg = edge_index_cg[0].astype(jnp.int32).reshape(ECG, 1)
    dcg = edge_index_cg[1].astype(jnp.int32).reshape(ECG, 1)
    b0r = b0.reshape(1, H)
    b1r = b1.reshape(1, H)
    bgr = bg.reshape(1, FC)
    bfcr = bfc.reshape(1, FC)
    attsr = att_src.reshape(1, FC)
    attdr = att_dst.reshape(1, FC)
    wfa = Wfc[:FC]
    wfb = Wfc[FC:]

    deg_k, mp_k = _make_sc_kernels()

    degp = deg_k(dstp, zeros_dg, ones_dg).reshape(NC, NP, 1)

    h0p = pl.pallas_call(
        _stage2_body,
        grid=(GRID_N,),
        in_specs=[_row_spec(DIN), _fixed_spec((DIN, H)), _DEGP_SPEC],
        out_specs=_row_spec(H),
        out_shape=jax.ShapeDtypeStruct((NP, H), jnp.float32),
    )(xp, W0, degp)

    acc1 = mp_k(h0p, srcp, dstp, zeros_mp)

    h1, h1p = pl.pallas_call(
        _stage4_body,
        grid=(GRID_N,),
        in_specs=[_ACC_SPEC, _row_spec(H), _DEGP_SPEC,
                  _fixed_spec((1, H)), _fixed_spec((H, H))],
        out_specs=[_row_spec(H), _row_spec(H)],
        out_shape=[jax.ShapeDtypeStruct((NP, H), jnp.float32),
                   jax.ShapeDtypeStruct((NP, H), jnp.float32)],
    )(acc1, h0p, degp, b0r, W1)

    acc2 = mp_k(h1p, srcp, dstp, zeros_mp)

    pooled = pl.pallas_call(
        _stage6_body,
        grid=(GRID_N,),
        in_specs=[_row_spec(DIN), _row_spec(H), _row_spec(H), _ACC_SPEC,
                  _DEGP_SPEC, _fixed_spec((1, H)), _row_spec(1)],
        out_specs=_fixed_spec((GP, FC)),
        out_shape=jax.ShapeDtypeStruct((GP, FC), jnp.float32),
    )(xp, h1, h1p, acc2, degp, b1r, batchp)

    outp = pl.pallas_call(
        _stage7_body,
        out_shape=jax.ShapeDtypeStruct((GP, FC), jnp.float32),
    )(pooled, Wg, attsr, attdr, bgr, scg, dcg, wfa, wfb, bfcr)

    return outp[:G]
